# baseline (speedup 1.0000x reference)
import functools

import jax
import jax.numpy as jnp
from jax import lax
from jax.experimental import pallas as pl
from jax.experimental.pallas import tpu as pltpu
from jax.experimental.pallas import tpu_sc as plsc

NUM_USERS = 10000
NUM_POIS = 10000
BATCH = 4096

NC = 2
NS = 16
L = 16
NW = NC * NS
BPW = BATCH // NW
C = 128
NMAIN = NUM_POIS // C
TAIL = NUM_POIS - NMAIN * C
NEG = -1000000000.0
SYNC_DEBUG = False


def _splat(val):
    return jnp.full((L,), val, jnp.int32)


def _body(table, tail_pad, uid, g_hbm, out_t, idx_v, g_v, gb0, gb1, tb0, tb1,
          gsem0, gsem1, ssem0, ssem1):
    _ROW_IDX = [lax.iota(jnp.int32, L) + bg * L for bg in range(BPW // L)]
    wid = lax.axis_index("s") * NC + lax.axis_index("c")
    wb = pl.multiple_of(wid * BPW, BPW)

    pltpu.sync_copy(uid.at[pl.ds(wb, BPW)], idx_v)
    pltpu.sync_copy(g_hbm, g_v)

    def gather_main(t, gb, gsem):
        c0 = pl.multiple_of(t * C, C)
        return pltpu.make_async_copy(table.at[idx_v, pl.ds(c0, C)], gb, gsem)

    def gather_tail(gb, gsem):
        return pltpu.make_async_copy(tail_pad.at[idx_v], gb, gsem)

    def accum(gb, acc):
        def it(p, a):
            ps = _splat(p)
            return tuple(
                a[bg] + plsc.load_gather(gb, [_ROW_IDX[bg], ps])
                for bg in range(8)
            )

        return lax.fori_loop(0, C, it, acc)

    if SYNC_DEBUG:
        def pa_sync(t, acc):
            g = gather_main(t, gb0, gsem0)
            g.start()
            g.wait()
            return accum(gb0, acc)

        acc0 = tuple(jnp.zeros((L,), jnp.float32) for _ in range(8))
        acc = lax.fori_loop(0, NMAIN, pa_sync, acc0)
        gt = gather_tail(gb0, gsem0)
        gt.start()
        gt.wait()
        acc = accum(gb0, acc)
        scales = [jnp.where(a != 0.0, jnp.float32(0.001), jnp.float32(1.0))
                  for a in acc]

        def scatter_main2(t, tb, ssem):
            c0 = pl.multiple_of(t * C, C)
            return pltpu.make_async_copy(
                tb, out_t.at[pl.ds(c0, C), pl.ds(wb, BPW)], ssem
            )

        def compute2(gb, tb, c0, np_count):
            def it(p, _):
                gs = plsc.load_gather(g_v, [_splat(c0 + p)])
                ps = _splat(p)
                for bg in range(8):
                    v = plsc.load_gather(gb, [_ROW_IDX[bg], ps])
                    tb[p, pl.ds(bg * L, L)] = v + gs * scales[bg]
                return 0

            lax.fori_loop(0, np_count, it, 0)

        def pb_sync(t, _):
            g = gather_main(t, gb0, gsem0)
            g.start()
            g.wait()
            compute2(gb0, tb0, pl.multiple_of(t * C, C), C)
            s = scatter_main2(t, tb0, ssem0)
            s.start()
            s.wait()
            return 0

        lax.fori_loop(0, NMAIN, pb_sync, 0)
        gt = gather_tail(gb0, gsem0)
        gt.start()
        gt.wait()
        compute2(gb0, tb0, NMAIN * C, TAIL)
        ts = pltpu.make_async_copy(
            tb0.at[pl.ds(0, TAIL)],
            out_t.at[pl.ds(NMAIN * C, TAIL), pl.ds(wb, BPW)],
            ssem0,
        )
        ts.start()
        ts.wait()
        neg = jnp.full((L,), NEG, jnp.float32)
        for bg in range(8):
            tb0[0, pl.ds(bg * L, L)] = neg
        pltpu.sync_copy(tb0.at[pl.ds(0, 1)],
                        out_t.at[pl.ds(0, 1), pl.ds(wb, BPW)])
        return

    warm = gather_main(0, gb1, gsem1)
    warm.start()
    warm.wait()

    gather_main(0, gb0, gsem0).start()

    def pa_pair(i, acc):
        t0 = 2 * i
        gather_main(t0, gb0, gsem0).wait()
        gather_main(t0 + 1, gb1, gsem1).start()
        acc = accum(gb0, acc)

        @pl.when(t0 + 2 < NMAIN)
        def _n0():
            gather_main(t0 + 2, gb0, gsem0).start()

        @pl.when(t0 + 2 == NMAIN)
        def _n0t():
            gather_tail(gb0, gsem0).start()

        gather_main(t0 + 1, gb1, gsem1).wait()
        acc = accum(gb1, acc)
        return acc

    acc0 = tuple(jnp.zeros((L,), jnp.float32) for _ in range(8))
    acc = lax.fori_loop(0, NMAIN // 2, pa_pair, acc0)
    gather_tail(gb0, gsem0).wait()
    acc = accum(gb0, acc)

    scales = [jnp.where(a != 0.0, jnp.float32(0.001), jnp.float32(1.0))
              for a in acc]

    def scatter_main(t, tb, ssem):
        c0 = pl.multiple_of(t * C, C)
        return pltpu.make_async_copy(
            tb, out_t.at[pl.ds(c0, C), pl.ds(wb, BPW)], ssem
        )

    def compute(gb, tb, c0, np_count):
        def it(p, _):
            gs = plsc.load_gather(g_v, [_splat(c0 + p)])
            ps = _splat(p)
            for bg in range(8):
                v = plsc.load_gather(gb, [_ROW_IDX[bg], ps])
                tb[p, pl.ds(bg * L, L)] = v + gs * scales[bg]
            return 0

        lax.fori_loop(0, np_count, it, 0)

    gather_main(0, gb0, gsem0).start()
    gather_main(1, gb1, gsem1).start()

    def pb_pair(i, _):
        t0 = 2 * i
        gather_main(t0, gb0, gsem0).wait()

        @pl.when(i >= 1)
        def _w0():
            scatter_main(t0 - 2, tb0, ssem0).wait()

        compute(gb0, tb0, pl.multiple_of(t0 * C, C), C)
        scatter_main(t0, tb0, ssem0).start()

        @pl.when(t0 + 2 < NMAIN)
        def _n0():
            gather_main(t0 + 2, gb0, gsem0).start()

        @pl.when(t0 + 2 == NMAIN)
        def _n0t():
            gather_tail(gb0, gsem0).start()

        gather_main(t0 + 1, gb1, gsem1).wait()

        @pl.when(i >= 1)
        def _w1():
            scatter_main(t0 - 1, tb1, ssem1).wait()

        compute(gb1, tb1, pl.multiple_of((t0 + 1) * C, C), C)
        scatter_main(t0 + 1, tb1, ssem1).start()

        @pl.when(t0 + 3 < NMAIN)
        def _n1():
            gather_main(t0 + 3, gb1, gsem1).start()

        return 0

    lax.fori_loop(0, NMAIN // 2, pb_pair, 0)

    gather_tail(gb0, gsem0).wait()
    scatter_main(NMAIN - 2, tb0, ssem0).wait()
    compute(gb0, tb0, NMAIN * C, TAIL)
    tail_sc = pltpu.make_async_copy(
        tb0.at[pl.ds(0, TAIL)],
        out_t.at[pl.ds(NMAIN * C, TAIL), pl.ds(wb, BPW)],
        ssem0,
    )
    tail_sc.start()
    scatter_main(NMAIN - 1, tb1, ssem1).wait()
    tail_sc.wait()

    g0 = gather_main(0, gb0, gsem0)
    g0.start()
    g0.wait()
    compute(gb0, tb0, 0, C)
    s0 = scatter_main(0, tb0, ssem0)
    s0.start()
    s0.wait()

    neg = jnp.full((L,), NEG, jnp.float32)
    for bg in range(8):
        tb1[0, pl.ds(bg * L, L)] = neg
    pltpu.sync_copy(tb1.at[pl.ds(0, 1)], out_t.at[pl.ds(0, 1), pl.ds(wb, BPW)])


_sc_call = functools.partial(
    pl.kernel,
    out_type=jax.ShapeDtypeStruct((NUM_POIS, BATCH), jnp.float32),
    mesh=plsc.VectorSubcoreMesh(
        core_axis_name="c", subcore_axis_name="s", num_cores=NC, num_subcores=NS
    ),
    scratch_types=[
        pltpu.VMEM((BPW,), jnp.int32),
        pltpu.VMEM((NUM_POIS,), jnp.float32),
        pltpu.VMEM((BPW, C), jnp.float32),
        pltpu.VMEM((BPW, C), jnp.float32),
        pltpu.VMEM((C, BPW), jnp.float32),
        pltpu.VMEM((C, BPW), jnp.float32),
        pltpu.SemaphoreType.DMA,
        pltpu.SemaphoreType.DMA,
        pltpu.SemaphoreType.DMA,
        pltpu.SemaphoreType.DMA,
    ],
    compiler_params=pltpu.CompilerParams(
        needs_layout_passes=False, use_tc_tiling_on_sc=True
    ),
)(_body)


def kernel(user_id, global_poi_cnt, user_poi_cnt):
    uid = user_id.astype(jnp.int32)
    tail_pad = jnp.pad(
        user_poi_cnt[:, NMAIN * C :], ((0, 0), (0, C - TAIL))
    )
    out_t = _sc_call(user_poi_cnt, tail_pad, uid, global_poi_cnt)
    return out_t.T


# speedup vs baseline: 2.4625x; 2.4625x over previous
import functools

import jax
import jax.numpy as jnp
from jax import lax
from jax.experimental import pallas as pl
from jax.experimental.pallas import tpu as pltpu
from jax.experimental.pallas import tpu_sc as plsc

NUM_USERS = 10000
NUM_POIS = 10000
BATCH = 4096

NC = 2
NS = 16
L = 16
NW = NC * NS
BPW = BATCH // NW
G = 2
T = BPW // G
R = 4
VECS = NUM_POIS // L
UNROLL = 25
NEG = -1000000000.0


def _process_row(buf, s, r, g_v):
    def it(j, acc):
        for u in range(UNROLL):
            off = (j * UNROLL + u) * L
            v = buf[s, r, pl.ds(off, L)]
            acc = acc + v
            buf[s, r, pl.ds(off, L)] = v + g_v[pl.ds(off, L)] * 0.001
        return acc

    acc = lax.fori_loop(0, VECS // UNROLL, it, jnp.zeros((L,), jnp.float32))
    nonzero_lanes = plsc.all_reduce_population_count(acc != 0.0)
    cold = nonzero_lanes[0] == 0

    @pl.when(cold)
    def _cold():
        def cp(j, c):
            for u in range(UNROLL):
                off = (j * UNROLL + u) * L
                buf[s, r, pl.ds(off, L)] = g_v[pl.ds(off, L)]
            return c

        lax.fori_loop(0, VECS // UNROLL, cp, 0)

    lane = lax.iota(jnp.int32, L)
    v0 = buf[s, r, pl.ds(0, L)]
    buf[s, r, pl.ds(0, L)] = jnp.where(lane == 0, NEG, v0)


def _body(table, uid, g_hbm, out, idx_v, g_v, buf, *sems):
    gsems, ssems = sems[:R], sems[R:]
    wid = lax.axis_index("s") * NC + lax.axis_index("c")
    base = wid * BPW

    pltpu.sync_copy(uid.at[wid], idx_v)
    pltpu.sync_copy(g_hbm, g_v)

    def gather(t, slot):
        return pltpu.make_async_copy(
            table.at[idx_v.at[t]], buf.at[slot], gsems[slot]
        )

    class _Multi:
        def __init__(self, descs):
            self.descs = descs

        def start(self):
            for d in self.descs:
                d.start()

        def wait(self):
            for d in self.descs:
                d.wait()

    def scatter(t, slot):
        return _Multi([
            pltpu.make_async_copy(
                buf.at[slot, r],
                out.at[pl.ds((base + t * G + r) * NUM_POIS, NUM_POIS)],
                ssems[slot],
            )
            for r in range(G)
        ])

    for s in range(R - 1):
        gather(s, s).start()

    def block(tb, _):
        for s in range(R):
            t = tb * R + s
            gather(t, s).wait()
            for r in range(G):
                _process_row(buf, s, r, g_v)
            scatter(t, s).start()
            ps = (s + R - 1) % R

            @pl.when(t >= 1)
            def _drain():
                scatter(t - 1, ps).wait()

            @pl.when(t + R - 1 < T)
            def _next():
                gather(t + R - 1, ps).start()

        return 0

    lax.fori_loop(0, T // R, block, 0)
    scatter(T - 1, (T - 1) % R).wait()


_sc_call = functools.partial(
    pl.kernel,
    out_type=jax.ShapeDtypeStruct((BATCH * NUM_POIS,), jnp.float32),
    mesh=plsc.VectorSubcoreMesh(
        core_axis_name="c", subcore_axis_name="s", num_cores=NC, num_subcores=NS
    ),
    scratch_types=[
        pltpu.VMEM((T, G), jnp.int32),
        pltpu.VMEM((NUM_POIS,), jnp.float32),
        pltpu.VMEM((R, G, NUM_POIS), jnp.float32),
    ]
    + [pltpu.SemaphoreType.DMA] * (2 * R),
    compiler_params=pltpu.CompilerParams(
        needs_layout_passes=False, use_tc_tiling_on_sc=False
    ),
)(_body)


def kernel(user_id, global_poi_cnt, user_poi_cnt):
    uid = user_id.astype(jnp.int32).reshape(NW, T, G)
    flat = _sc_call(user_poi_cnt, uid, global_poi_cnt)
    return flat.reshape(BATCH, NUM_POIS)
